# baseline (speedup 1.0000x reference)
import functools

import jax
import jax.numpy as jnp
from jax import lax
from jax.experimental import pallas as pl
from jax.experimental.pallas import tpu as pltpu
from jax.experimental.pallas import tpu_sc as plsc

N = 10000
E = 320000
D = 128
DE = 16
NC = 2
NS = 16
NW = NC * NS
CHUNK = 64
IDXB = 4
GEDGE = IDXB * CHUNK
NG0 = 24
NG1 = 56
NGT = NS * (NG0 + NG1)
E_PAD = NGT * GEDGE
N_PAD = 10112
STRIPE = N_PAD // NS

_mesh = plsc.VectorSubcoreMesh(core_axis_name="c", subcore_axis_name="s")


@functools.partial(
    pl.kernel,
    out_type=(
        jax.ShapeDtypeStruct((NC, N_PAD, D), jnp.float32),
        jax.ShapeDtypeStruct((NC, N_PAD, DE), jnp.float32),
        jax.ShapeDtypeStruct((NC, N_PAD, 8), jnp.float32),
    ),
    mesh=_mesh,
    compiler_params=pltpu.CompilerParams(use_tc_tiling_on_sc=False),
    scratch_types=[
        pltpu.VMEM((IDXB, CHUNK), jnp.int32),
        pltpu.VMEM((IDXB, CHUNK), jnp.int32),
        pltpu.VMEM((2, CHUNK, D), jnp.float32),
        pltpu.VMEM((GEDGE, DE), jnp.float32),
        pltpu.VMEM((CHUNK, 8), jnp.float32),
        pltpu.VMEM_SHARED((N_PAD, D), jnp.float32),
        pltpu.VMEM_SHARED((N_PAD, DE), jnp.float32),
        pltpu.VMEM_SHARED((N_PAD, 8), jnp.float32),
        pltpu.SemaphoreType.DMA((2,)),
        pltpu.SemaphoreType.DMA((2,)),
        pltpu.SemaphoreType.DMA((2,)),
    ],
)
def _sc_agg1(x_hbm, ea_hbm, src_hbm, dst_hbm, ones_hbm, zx_hbm, ze_hbm, zd_hbm,
             ox_hbm, oe_hbm, od_hbm,
             src_v, dst_v, xrows, ea4, ones_v, accx, acce, accd,
             sem_g, sem_s, sem_e):
    cid = lax.axis_index("c")
    sid = lax.axis_index("s")

    r0 = sid * STRIPE
    pltpu.sync_copy(zx_hbm.at[pl.ds(r0, STRIPE)], accx.at[pl.ds(r0, STRIPE)])
    pltpu.sync_copy(ze_hbm.at[pl.ds(r0, STRIPE)], acce.at[pl.ds(r0, STRIPE)])
    pltpu.sync_copy(zd_hbm.at[pl.ds(r0, STRIPE)], accd.at[pl.ds(r0, STRIPE)])
    pltpu.sync_copy(ones_hbm, ones_v)
    plsc.subcore_barrier()

    ng = NG0 + cid * (NG1 - NG0)
    gb = cid * (NS * NG0) + sid * ng

    def grp(g, carry):
        pltpu.sync_copy(src_hbm.at[gb + g], src_v)
        pltpu.sync_copy(dst_hbm.at[gb + g], dst_v)
        goff = (gb + g) * GEDGE
        pltpu.sync_copy(ea_hbm.at[pl.ds(goff, GEDGE)], ea4)

        eds = []
        for jj in range(IDXB):
            didx = dst_v.at[jj]
            eds.append(pltpu.async_copy(
                ea4.at[pl.ds(jj * CHUNK, CHUNK)], acce.at[didx],
                sem_e.at[0], add=True))
            eds.append(pltpu.async_copy(
                ones_v, accd.at[didx], sem_e.at[1], add=True))

        def G(jj):
            return pltpu.async_copy(x_hbm.at[src_v.at[jj]],
                                    xrows.at[jj % 2], sem_g.at[jj % 2])

        def S(jj):
            return pltpu.async_copy(xrows.at[jj % 2], accx.at[dst_v.at[jj]],
                                    sem_s.at[jj % 2], add=True)

        gd0 = G(0)
        gd1 = G(1)
        gd0.wait(); sx0 = S(0)
        gd1.wait(); sx1 = S(1)
        sx0.wait(); gd2 = G(2)
        sx1.wait(); gd3 = G(3)
        gd2.wait(); sx2 = S(2)
        gd3.wait(); sx3 = S(3)
        sx2.wait(); sx3.wait()
        for d in eds:
            d.wait()
        return carry

    lax.fori_loop(0, ng, grp, 0)
    plsc.subcore_barrier()
    pltpu.sync_copy(accx.at[pl.ds(r0, STRIPE)], ox_hbm.at[cid, pl.ds(r0, STRIPE)])
    pltpu.sync_copy(acce.at[pl.ds(r0, STRIPE)], oe_hbm.at[cid, pl.ds(r0, STRIPE)])
    pltpu.sync_copy(accd.at[pl.ds(r0, STRIPE)], od_hbm.at[cid, pl.ds(r0, STRIPE)])


@functools.partial(
    pl.kernel,
    out_type=jax.ShapeDtypeStruct((NC, N_PAD, D), jnp.float32),
    mesh=_mesh,
    compiler_params=pltpu.CompilerParams(use_tc_tiling_on_sc=False),
    scratch_types=[
        pltpu.VMEM((IDXB, CHUNK), jnp.int32),
        pltpu.VMEM((IDXB, CHUNK), jnp.int32),
        pltpu.VMEM((2, CHUNK, D), jnp.float32),
        pltpu.VMEM_SHARED((N_PAD, D), jnp.float32),
        pltpu.SemaphoreType.DMA((2,)),
        pltpu.SemaphoreType.DMA((2,)),
    ],
)
def _sc_agg2(h_hbm, src_hbm, dst_hbm, zx_hbm, oh_hbm,
             src_v, dst_v, hrows, acch, sem_g, sem_s):
    cid = lax.axis_index("c")
    sid = lax.axis_index("s")

    r0 = sid * STRIPE
    pltpu.sync_copy(zx_hbm.at[pl.ds(r0, STRIPE)], acch.at[pl.ds(r0, STRIPE)])
    plsc.subcore_barrier()

    ng = NG0 + cid * (NG1 - NG0)
    gb = cid * (NS * NG0) + sid * ng

    def grp(g, carry):
        pltpu.sync_copy(src_hbm.at[gb + g], src_v)
        pltpu.sync_copy(dst_hbm.at[gb + g], dst_v)

        def G(jj):
            return pltpu.async_copy(h_hbm.at[src_v.at[jj]],
                                    hrows.at[jj % 2], sem_g.at[jj % 2])

        def S(jj):
            return pltpu.async_copy(hrows.at[jj % 2], acch.at[dst_v.at[jj]],
                                    sem_s.at[jj % 2], add=True)

        gd0 = G(0)
        gd1 = G(1)
        gd0.wait(); sx0 = S(0)
        gd1.wait(); sx1 = S(1)
        sx0.wait(); gd2 = G(2)
        sx1.wait(); gd3 = G(3)
        gd2.wait(); sx2 = S(2)
        gd3.wait(); sx3 = S(3)
        sx2.wait(); sx3.wait()
        return carry

    lax.fori_loop(0, ng, grp, 0)
    plsc.subcore_barrier()
    pltpu.sync_copy(acch.at[pl.ds(r0, STRIPE)], oh_hbm.at[cid, pl.ds(r0, STRIPE)])


@functools.partial(
    pl.kernel,
    out_type=jax.ShapeDtypeStruct((E_PAD, 8), jnp.float32),
    mesh=_mesh,
    compiler_params=pltpu.CompilerParams(use_tc_tiling_on_sc=False),
    scratch_types=[
        pltpu.VMEM((IDXB, CHUNK), jnp.int32),
        pltpu.VMEM((IDXB, CHUNK), jnp.int32),
        pltpu.VMEM((IDXB, CHUNK, 8), jnp.float32),
        pltpu.SemaphoreType.DMA((2,)),
    ],
)
def _sc_edge_score(ps_hbm, pd_hbm, src_hbm, dst_hbm, out_hbm,
                   src_v, dst_v, s_v, sem):
    cid = lax.axis_index("c")
    sid = lax.axis_index("s")
    ng = NG0 + cid * (NG1 - NG0)
    gb = cid * (NS * NG0) + sid * ng

    def grp(g, carry):
        pltpu.sync_copy(src_hbm.at[gb + g], src_v)
        pltpu.sync_copy(dst_hbm.at[gb + g], dst_v)
        goff = (gb + g) * GEDGE

        ds = [pltpu.async_copy(ps_hbm.at[src_v.at[jj]], s_v.at[jj],
                               sem.at[0]) for jj in range(IDXB)]
        das = []
        for jj in range(IDXB):
            ds[jj].wait()
            das.append(pltpu.async_copy(pd_hbm.at[dst_v.at[jj]], s_v.at[jj],
                                        sem.at[1], add=True))
        for d in das:
            d.wait()
        for jj in range(IDXB):
            pltpu.sync_copy(s_v.at[jj], out_hbm.at[pl.ds(goff + jj * CHUNK, CHUNK)])
        return carry

    lax.fori_loop(0, ng, grp, 0)


CBLK = 2048
CROWS = E_PAD // 16


def _tc_compact(s8, sel):
    def body(sr, selr, outr):
        outr[...] = jnp.dot(sr[...], selr[...],
                            preferred_element_type=jnp.float32)

    return pl.pallas_call(
        body,
        grid=(CROWS // CBLK,),
        in_specs=[pl.BlockSpec((CBLK, 128), lambda i: (i, 0)),
                  pl.BlockSpec((128, 32), lambda i: (0, 0))],
        out_specs=pl.BlockSpec((CBLK, 32), lambda i: (i, 0)),
        out_shape=jax.ShapeDtypeStruct((CROWS, 32), jnp.float32),
    )(s8, sel)


RB = 632
NRB = N_PAD // RB


def _row_spec(c):
    return pl.BlockSpec((RB, c), lambda i: (i, 0))


def _block_spec(r, c):
    return pl.BlockSpec((r, c), lambda i: (0, 0))


def _tc_layer(ox0, ox1, oe0, oe1, od0, od1, h, wmh, wme, bm, wah, wan, ba,
              wp=None, bp=None):
    with_p = wp is not None
    if not with_p:
        wp = jnp.zeros((D, 16), jnp.float32)
        bp = jnp.zeros((1, 8), jnp.float32)

    def body(ox0r, ox1r, oe0r, oe1r, od0r, od1r, hr, wmhr, wmer, bmr, wahr,
             wanr, bar, wpr, bpr, hor, *maybe_p):
        aggh = ox0r[...] + ox1r[...]
        agge = oe0r[...] + oe1r[...]
        deg = od0r[...][:, 0:1] + od1r[...][:, 0:1]
        s = (jnp.dot(aggh, wmhr[...], preferred_element_type=jnp.float32)
             + jnp.dot(agge, wmer[...], preferred_element_type=jnp.float32)
             + deg * bmr[...])
        hn = jnp.where(deg > 0, s / jnp.maximum(deg, 1.0), 0.0)
        hnew = jax.nn.relu(
            jnp.dot(hr[...], wahr[...], preferred_element_type=jnp.float32)
            + jnp.dot(hn, wanr[...], preferred_element_type=jnp.float32)
            + bar[...])
        hor[...] = hnew
        if maybe_p:
            p = jnp.dot(hnew, wpr[...], preferred_element_type=jnp.float32)
            maybe_p[0][...] = p[:, 0:8]
            maybe_p[1][...] = p[:, 8:16] + bpr[...]

    out_shape = [jax.ShapeDtypeStruct((N_PAD, D), jnp.float32)]
    out_specs = [_row_spec(D)]
    if with_p:
        out_shape += [jax.ShapeDtypeStruct((N_PAD, 8), jnp.float32)] * 2
        out_specs += [_row_spec(8)] * 2

    res = pl.pallas_call(
        body,
        grid=(NRB,),
        in_specs=[
            _row_spec(D), _row_spec(D),
            _row_spec(DE), _row_spec(DE),
            _row_spec(8), _row_spec(8),
            _row_spec(D),
            _block_spec(D, D), _block_spec(DE, D), _block_spec(1, D),
            _block_spec(D, D), _block_spec(D, D), _block_spec(1, D),
            _block_spec(D, 16), _block_spec(1, 8),
        ],
        out_specs=out_specs,
        out_shape=out_shape,
    )(ox0, ox1, oe0, oe1, od0, od1, h, wmh, wme, bm, wah, wan, ba, wp, bp)
    return res if with_p else res[0]


def kernel(x, edge_index, edge_attr, W_msg1, b_msg1, W_apply1, b_apply1,
           W_msg2, b_msg2, W_apply2, b_apply2, W_pred, b_pred):
    src = edge_index[0].astype(jnp.int32)
    dst = edge_index[1].astype(jnp.int32)

    pad = E_PAD - E
    src_p = jnp.concatenate([src, jnp.zeros((pad,), jnp.int32)]
                            ).reshape(NGT, IDXB, CHUNK)
    dst_p = jnp.concatenate([dst, jnp.full((pad,), N, jnp.int32)]
                            ).reshape(NGT, IDXB, CHUNK)
    ea_p = jnp.concatenate([edge_attr, jnp.zeros((pad, DE), jnp.float32)])

    x_p = jnp.concatenate([x, jnp.zeros((N_PAD - N, D), jnp.float32)])
    ones8 = jnp.ones((CHUNK, 8), jnp.float32)
    zx = jnp.zeros((N_PAD, D), jnp.float32)
    ze = jnp.zeros((N_PAD, DE), jnp.float32)
    zd = jnp.zeros((N_PAD, 8), jnp.float32)

    ox, oe, od = _sc_agg1(x_p, ea_p, src_p, dst_p, ones8, zx, ze, zd)

    h1 = _tc_layer(ox[0], ox[1], oe[0], oe[1], od[0], od[1], x_p,
                   W_msg1[:D], W_msg1[D:], b_msg1[None, :],
                   W_apply1[:D], W_apply1[D:], b_apply1[None, :])

    oh = _sc_agg2(h1, src_p, dst_p, zx)

    wp16 = jnp.zeros((D, 16), jnp.float32)
    wp16 = wp16.at[:, 0:2].set(W_pred[:D]).at[:, 8:10].set(W_pred[D:])
    bp8 = jnp.zeros((1, 8), jnp.float32).at[0, 0:2].set(b_pred)

    _, ps_tab, pd_tab = _tc_layer(
        oh[0], oh[1], oe[0], oe[1], od[0], od[1], h1,
        W_msg2[:D], W_msg2[D:], b_msg2[None, :],
        W_apply2[:D], W_apply2[D:], b_apply2[None, :],
        wp=wp16, bp=bp8)

    s8 = _sc_edge_score(ps_tab, pd_tab, src_p, dst_p)
    sel = jnp.zeros((128, 32), jnp.float32)
    ke = jnp.arange(16)
    for c in range(2):
        sel = sel.at[8 * ke + c, 2 * ke + c].set(1.0)
    out32 = _tc_compact(s8.reshape(CROWS, 128), sel)
    return out32.reshape(E_PAD, 2)[:E]


# speedup vs baseline: 1.1009x; 1.1009x over previous
import functools

import jax
import jax.numpy as jnp
from jax import lax
from jax.experimental import pallas as pl
from jax.experimental.pallas import tpu as pltpu
from jax.experimental.pallas import tpu_sc as plsc

N = 10000
E = 320000
D = 128
DE = 16
NC = 2
NS = 16
NW = NC * NS
CHUNK = 64
IDXB = 4
GEDGE = IDXB * CHUNK
NG0 = 40
NG1 = 40
NGT = NS * (NG0 + NG1)
E_PAD = NGT * GEDGE
N_PAD = 10112
STRIPE = N_PAD // NS

_mesh = plsc.VectorSubcoreMesh(core_axis_name="c", subcore_axis_name="s")


@functools.partial(
    pl.kernel,
    out_type=(
        jax.ShapeDtypeStruct((NC, N_PAD, D), jnp.float32),
        jax.ShapeDtypeStruct((NC, N_PAD, DE), jnp.float32),
        jax.ShapeDtypeStruct((NC, N_PAD, 8), jnp.float32),
    ),
    mesh=_mesh,
    compiler_params=pltpu.CompilerParams(use_tc_tiling_on_sc=False),
    scratch_types=[
        pltpu.VMEM((IDXB, CHUNK), jnp.int32),
        pltpu.VMEM((IDXB, CHUNK), jnp.int32),
        pltpu.VMEM((2, CHUNK, D), jnp.float32),
        pltpu.VMEM((GEDGE, DE), jnp.float32),
        pltpu.VMEM((CHUNK, 8), jnp.float32),
        pltpu.VMEM_SHARED((N_PAD, D), jnp.float32),
        pltpu.VMEM_SHARED((N_PAD, DE), jnp.float32),
        pltpu.VMEM_SHARED((N_PAD, 8), jnp.float32),
        pltpu.SemaphoreType.DMA((2,)),
        pltpu.SemaphoreType.DMA((2,)),
        pltpu.SemaphoreType.DMA((2,)),
    ],
)
def _sc_agg1(x_hbm, ea_hbm, src_hbm, dst_hbm, ones_hbm, zx_hbm, ze_hbm, zd_hbm,
             ox_hbm, oe_hbm, od_hbm,
             src_v, dst_v, xrows, ea4, ones_v, accx, acce, accd,
             sem_g, sem_s, sem_e):
    cid = lax.axis_index("c")
    sid = lax.axis_index("s")

    r0 = sid * STRIPE
    pltpu.sync_copy(zx_hbm.at[pl.ds(r0, STRIPE)], accx.at[pl.ds(r0, STRIPE)])
    pltpu.sync_copy(ze_hbm.at[pl.ds(r0, STRIPE)], acce.at[pl.ds(r0, STRIPE)])
    pltpu.sync_copy(zd_hbm.at[pl.ds(r0, STRIPE)], accd.at[pl.ds(r0, STRIPE)])
    pltpu.sync_copy(ones_hbm, ones_v)
    plsc.subcore_barrier()

    ng = NG0 + cid * (NG1 - NG0)
    gb = cid * (NS * NG0) + sid * ng

    def grp(g, carry):
        pltpu.sync_copy(src_hbm.at[gb + g], src_v)
        pltpu.sync_copy(dst_hbm.at[gb + g], dst_v)
        goff = (gb + g) * GEDGE
        pltpu.sync_copy(ea_hbm.at[pl.ds(goff, GEDGE)], ea4)

        eds = []
        for jj in range(IDXB):
            didx = dst_v.at[jj]
            eds.append(pltpu.async_copy(
                ea4.at[pl.ds(jj * CHUNK, CHUNK)], acce.at[didx],
                sem_e.at[0], add=True))
            eds.append(pltpu.async_copy(
                ones_v, accd.at[didx], sem_e.at[1], add=True))

        def G(jj):
            return pltpu.async_copy(x_hbm.at[src_v.at[jj]],
                                    xrows.at[jj % 2], sem_g.at[jj % 2])

        def S(jj):
            return pltpu.async_copy(xrows.at[jj % 2], accx.at[dst_v.at[jj]],
                                    sem_s.at[jj % 2], add=True)

        gd0 = G(0)
        gd1 = G(1)
        gd0.wait(); sx0 = S(0)
        gd1.wait(); sx1 = S(1)
        sx0.wait(); gd2 = G(2)
        sx1.wait(); gd3 = G(3)
        gd2.wait(); sx2 = S(2)
        gd3.wait(); sx3 = S(3)
        sx2.wait(); sx3.wait()
        for d in eds:
            d.wait()
        return carry

    lax.fori_loop(0, ng, grp, 0)
    plsc.subcore_barrier()
    pltpu.sync_copy(accx.at[pl.ds(r0, STRIPE)], ox_hbm.at[cid, pl.ds(r0, STRIPE)])
    pltpu.sync_copy(acce.at[pl.ds(r0, STRIPE)], oe_hbm.at[cid, pl.ds(r0, STRIPE)])
    pltpu.sync_copy(accd.at[pl.ds(r0, STRIPE)], od_hbm.at[cid, pl.ds(r0, STRIPE)])


@functools.partial(
    pl.kernel,
    out_type=jax.ShapeDtypeStruct((NC, N_PAD, D), jnp.float32),
    mesh=_mesh,
    compiler_params=pltpu.CompilerParams(use_tc_tiling_on_sc=False),
    scratch_types=[
        pltpu.VMEM((IDXB, CHUNK), jnp.int32),
        pltpu.VMEM((IDXB, CHUNK), jnp.int32),
        pltpu.VMEM((2, CHUNK, D), jnp.float32),
        pltpu.VMEM_SHARED((N_PAD, D), jnp.float32),
        pltpu.SemaphoreType.DMA((2,)),
        pltpu.SemaphoreType.DMA((2,)),
    ],
)
def _sc_agg2(h_hbm, src_hbm, dst_hbm, zx_hbm, oh_hbm,
             src_v, dst_v, hrows, acch, sem_g, sem_s):
    cid = lax.axis_index("c")
    sid = lax.axis_index("s")

    r0 = sid * STRIPE
    pltpu.sync_copy(zx_hbm.at[pl.ds(r0, STRIPE)], acch.at[pl.ds(r0, STRIPE)])
    plsc.subcore_barrier()

    ng = NG0 + cid * (NG1 - NG0)
    gb = cid * (NS * NG0) + sid * ng

    def grp(g, carry):
        pltpu.sync_copy(src_hbm.at[gb + g], src_v)
        pltpu.sync_copy(dst_hbm.at[gb + g], dst_v)

        def G(jj):
            return pltpu.async_copy(h_hbm.at[src_v.at[jj]],
                                    hrows.at[jj % 2], sem_g.at[jj % 2])

        def S(jj):
            return pltpu.async_copy(hrows.at[jj % 2], acch.at[dst_v.at[jj]],
                                    sem_s.at[jj % 2], add=True)

        gd0 = G(0)
        gd1 = G(1)
        gd0.wait(); sx0 = S(0)
        gd1.wait(); sx1 = S(1)
        sx0.wait(); gd2 = G(2)
        sx1.wait(); gd3 = G(3)
        gd2.wait(); sx2 = S(2)
        gd3.wait(); sx3 = S(3)
        sx2.wait(); sx3.wait()
        return carry

    lax.fori_loop(0, ng, grp, 0)
    plsc.subcore_barrier()
    pltpu.sync_copy(acch.at[pl.ds(r0, STRIPE)], oh_hbm.at[cid, pl.ds(r0, STRIPE)])


@functools.partial(
    pl.kernel,
    out_type=jax.ShapeDtypeStruct((E_PAD, 8), jnp.float32),
    mesh=_mesh,
    compiler_params=pltpu.CompilerParams(use_tc_tiling_on_sc=False),
    scratch_types=[
        pltpu.VMEM((IDXB, CHUNK), jnp.int32),
        pltpu.VMEM((IDXB, CHUNK), jnp.int32),
        pltpu.VMEM((IDXB, CHUNK, 8), jnp.float32),
        pltpu.SemaphoreType.DMA((2,)),
    ],
)
def _sc_edge_score(ps_hbm, pd_hbm, src_hbm, dst_hbm, out_hbm,
                   src_v, dst_v, s_v, sem):
    cid = lax.axis_index("c")
    sid = lax.axis_index("s")
    ng = NG0 + cid * (NG1 - NG0)
    gb = cid * (NS * NG0) + sid * ng

    def grp(g, carry):
        pltpu.sync_copy(src_hbm.at[gb + g], src_v)
        pltpu.sync_copy(dst_hbm.at[gb + g], dst_v)
        goff = (gb + g) * GEDGE

        ds = [pltpu.async_copy(ps_hbm.at[src_v.at[jj]], s_v.at[jj],
                               sem.at[0]) for jj in range(IDXB)]
        das = []
        for jj in range(IDXB):
            ds[jj].wait()
            das.append(pltpu.async_copy(pd_hbm.at[dst_v.at[jj]], s_v.at[jj],
                                        sem.at[1], add=True))
        for d in das:
            d.wait()
        for jj in range(IDXB):
            pltpu.sync_copy(s_v.at[jj], out_hbm.at[pl.ds(goff + jj * CHUNK, CHUNK)])
        return carry

    lax.fori_loop(0, ng, grp, 0)


CBLK = 2048
CROWS = E_PAD // 16


def _tc_compact(s8, sel):
    def body(sr, selr, outr):
        outr[...] = jnp.dot(sr[...], selr[...],
                            preferred_element_type=jnp.float32)

    return pl.pallas_call(
        body,
        grid=(CROWS // CBLK,),
        in_specs=[pl.BlockSpec((CBLK, 128), lambda i: (i, 0)),
                  pl.BlockSpec((128, 32), lambda i: (0, 0))],
        out_specs=pl.BlockSpec((CBLK, 32), lambda i: (i, 0)),
        out_shape=jax.ShapeDtypeStruct((CROWS, 32), jnp.float32),
    )(s8, sel)


RB = 632
NRB = N_PAD // RB


def _row_spec(c):
    return pl.BlockSpec((RB, c), lambda i: (i, 0))


def _block_spec(r, c):
    return pl.BlockSpec((r, c), lambda i: (0, 0))


def _tc_layer(ox0, ox1, oe0, oe1, od0, od1, h, wmh, wme, bm, wah, wan, ba,
              wp=None, bp=None):
    with_p = wp is not None
    if not with_p:
        wp = jnp.zeros((D, 16), jnp.float32)
        bp = jnp.zeros((1, 8), jnp.float32)

    def body(ox0r, ox1r, oe0r, oe1r, od0r, od1r, hr, wmhr, wmer, bmr, wahr,
             wanr, bar, wpr, bpr, hor, *maybe_p):
        aggh = ox0r[...] + ox1r[...]
        agge = oe0r[...] + oe1r[...]
        deg = od0r[...][:, 0:1] + od1r[...][:, 0:1]
        s = (jnp.dot(aggh, wmhr[...], preferred_element_type=jnp.float32)
             + jnp.dot(agge, wmer[...], preferred_element_type=jnp.float32)
             + deg * bmr[...])
        hn = jnp.where(deg > 0, s / jnp.maximum(deg, 1.0), 0.0)
        hnew = jax.nn.relu(
            jnp.dot(hr[...], wahr[...], preferred_element_type=jnp.float32)
            + jnp.dot(hn, wanr[...], preferred_element_type=jnp.float32)
            + bar[...])
        hor[...] = hnew
        if maybe_p:
            p = jnp.dot(hnew, wpr[...], preferred_element_type=jnp.float32)
            maybe_p[0][...] = p[:, 0:8]
            maybe_p[1][...] = p[:, 8:16] + bpr[...]

    out_shape = [jax.ShapeDtypeStruct((N_PAD, D), jnp.float32)]
    out_specs = [_row_spec(D)]
    if with_p:
        out_shape += [jax.ShapeDtypeStruct((N_PAD, 8), jnp.float32)] * 2
        out_specs += [_row_spec(8)] * 2

    res = pl.pallas_call(
        body,
        grid=(NRB,),
        in_specs=[
            _row_spec(D), _row_spec(D),
            _row_spec(DE), _row_spec(DE),
            _row_spec(8), _row_spec(8),
            _row_spec(D),
            _block_spec(D, D), _block_spec(DE, D), _block_spec(1, D),
            _block_spec(D, D), _block_spec(D, D), _block_spec(1, D),
            _block_spec(D, 16), _block_spec(1, 8),
        ],
        out_specs=out_specs,
        out_shape=out_shape,
    )(ox0, ox1, oe0, oe1, od0, od1, h, wmh, wme, bm, wah, wan, ba, wp, bp)
    return res if with_p else res[0]


def kernel(x, edge_index, edge_attr, W_msg1, b_msg1, W_apply1, b_apply1,
           W_msg2, b_msg2, W_apply2, b_apply2, W_pred, b_pred):
    src = edge_index[0].astype(jnp.int32)
    dst = edge_index[1].astype(jnp.int32)

    pad = E_PAD - E
    src_p = jnp.concatenate([src, jnp.zeros((pad,), jnp.int32)]
                            ).reshape(NGT, IDXB, CHUNK)
    dst_p = jnp.concatenate([dst, jnp.full((pad,), N, jnp.int32)]
                            ).reshape(NGT, IDXB, CHUNK)
    ea_p = jnp.concatenate([edge_attr, jnp.zeros((pad, DE), jnp.float32)])

    x_p = jnp.concatenate([x, jnp.zeros((N_PAD - N, D), jnp.float32)])
    ones8 = jnp.ones((CHUNK, 8), jnp.float32)
    zx = jnp.zeros((N_PAD, D), jnp.float32)
    ze = jnp.zeros((N_PAD, DE), jnp.float32)
    zd = jnp.zeros((N_PAD, 8), jnp.float32)

    ox, oe, od = _sc_agg1(x_p, ea_p, src_p, dst_p, ones8, zx, ze, zd)

    h1 = _tc_layer(ox[0], ox[1], oe[0], oe[1], od[0], od[1], x_p,
                   W_msg1[:D], W_msg1[D:], b_msg1[None, :],
                   W_apply1[:D], W_apply1[D:], b_apply1[None, :])

    oh = _sc_agg2(h1, src_p, dst_p, zx)

    wp16 = jnp.zeros((D, 16), jnp.float32)
    wp16 = wp16.at[:, 0:2].set(W_pred[:D]).at[:, 8:10].set(W_pred[D:])
    bp8 = jnp.zeros((1, 8), jnp.float32).at[0, 0:2].set(b_pred)

    _, ps_tab, pd_tab = _tc_layer(
        oh[0], oh[1], oe[0], oe[1], od[0], od[1], h1,
        W_msg2[:D], W_msg2[D:], b_msg2[None, :],
        W_apply2[:D], W_apply2[D:], b_apply2[None, :],
        wp=wp16, bp=bp8)

    s8 = _sc_edge_score(ps_tab, pd_tab, src_p, dst_p)
    sel = jnp.zeros((128, 32), jnp.float32)
    ke = jnp.arange(16)
    for c in range(2):
        sel = sel.at[8 * ke + c, 2 * ke + c].set(1.0)
    out32 = _tc_compact(s8.reshape(CROWS, 128), sel)
    return out32.reshape(E_PAD, 2)[:E]


# speedup vs baseline: 1.3759x; 1.2498x over previous
import functools

import jax
import jax.numpy as jnp
from jax import lax
from jax.experimental import pallas as pl
from jax.experimental.pallas import tpu as pltpu
from jax.experimental.pallas import tpu_sc as plsc

N = 10000
E = 320000
D = 128
DE = 16
NC = 2
NS = 16
NW = NC * NS
CHUNK = 64
IDXB = 4
GEDGE = IDXB * CHUNK
NGT = E // GEDGE
S_FULL = 632
S_LAST = N - 15 * S_FULL

_mesh = plsc.VectorSubcoreMesh(core_axis_name="c", subcore_axis_name="s")
_sc_params = pltpu.CompilerParams(use_tc_tiling_on_sc=False)


def _grange(cid, sid):
    w = cid * NS + sid
    gb = (w * NGT) // NW
    ge = ((w + 1) * NGT) // NW
    return gb, ge


def _stripes(copy_fn):
    sid = lax.axis_index("s")

    @pl.when(sid < 15)
    def _():
        copy_fn(sid * S_FULL, S_FULL)

    @pl.when(sid == 15)
    def _():
        copy_fn(15 * S_FULL, S_LAST)


@functools.partial(
    pl.kernel,
    out_type=(
        jax.ShapeDtypeStruct((NC, N, D), jnp.float32),
        jax.ShapeDtypeStruct((NC, N, DE), jnp.float32),
        jax.ShapeDtypeStruct((NC, N, 8), jnp.float32),
    ),
    mesh=_mesh,
    compiler_params=_sc_params,
    scratch_types=[
        pltpu.VMEM((IDXB, CHUNK), jnp.int32),
        pltpu.VMEM((IDXB, CHUNK), jnp.int32),
        pltpu.VMEM((2, CHUNK, D), jnp.float32),
        pltpu.VMEM((GEDGE, DE), jnp.float32),
        pltpu.VMEM((CHUNK, 8), jnp.float32),
        pltpu.VMEM_SHARED((N, D), jnp.float32),
        pltpu.VMEM_SHARED((N, DE), jnp.float32),
        pltpu.VMEM_SHARED((N, 8), jnp.float32),
        pltpu.SemaphoreType.DMA((2,)),
        pltpu.SemaphoreType.DMA((2,)),
        pltpu.SemaphoreType.DMA((2,)),
    ],
)
def _sc_agg1(x_hbm, ea_hbm, src_hbm, dst_hbm, ones_hbm, zx_hbm, ze_hbm, zd_hbm,
             ox_hbm, oe_hbm, od_hbm,
             src_v, dst_v, xrows, ea4, ones_v, accx, acce, accd,
             sem_g, sem_s, sem_e):
    cid = lax.axis_index("c")
    sid = lax.axis_index("s")

    def zinit(r0, nr):
        pltpu.sync_copy(zx_hbm.at[pl.ds(r0, nr)], accx.at[pl.ds(r0, nr)])
        pltpu.sync_copy(ze_hbm.at[pl.ds(r0, nr)], acce.at[pl.ds(r0, nr)])
        pltpu.sync_copy(zd_hbm.at[pl.ds(r0, nr)], accd.at[pl.ds(r0, nr)])

    _stripes(zinit)
    pltpu.sync_copy(ones_hbm, ones_v)
    plsc.subcore_barrier()

    gb, ge = _grange(cid, sid)

    def grp(g, carry):
        pltpu.sync_copy(src_hbm.at[g], src_v)
        pltpu.sync_copy(dst_hbm.at[g], dst_v)
        goff = g * GEDGE
        pltpu.sync_copy(ea_hbm.at[pl.ds(goff, GEDGE)], ea4)

        eds = []
        for jj in range(IDXB):
            didx = dst_v.at[jj]
            eds.append(pltpu.async_copy(
                ea4.at[pl.ds(jj * CHUNK, CHUNK)], acce.at[didx],
                sem_e.at[0], add=True))
            eds.append(pltpu.async_copy(
                ones_v, accd.at[didx], sem_e.at[1], add=True))

        def G(jj):
            return pltpu.async_copy(x_hbm.at[src_v.at[jj]],
                                    xrows.at[jj % 2], sem_g.at[jj % 2])

        def S(jj):
            return pltpu.async_copy(xrows.at[jj % 2], accx.at[dst_v.at[jj]],
                                    sem_s.at[jj % 2], add=True)

        gd0 = G(0)
        gd1 = G(1)
        gd0.wait(); sx0 = S(0)
        gd1.wait(); sx1 = S(1)
        sx0.wait(); gd2 = G(2)
        sx1.wait(); gd3 = G(3)
        gd2.wait(); sx2 = S(2)
        gd3.wait(); sx3 = S(3)
        sx2.wait(); sx3.wait()
        for d in eds:
            d.wait()
        return carry

    lax.fori_loop(gb, ge, grp, 0)
    plsc.subcore_barrier()

    def wback(r0, nr):
        pltpu.sync_copy(accx.at[pl.ds(r0, nr)], ox_hbm.at[cid, pl.ds(r0, nr)])
        pltpu.sync_copy(acce.at[pl.ds(r0, nr)], oe_hbm.at[cid, pl.ds(r0, nr)])
        pltpu.sync_copy(accd.at[pl.ds(r0, nr)], od_hbm.at[cid, pl.ds(r0, nr)])

    _stripes(wback)


@functools.partial(
    pl.kernel,
    out_type=jax.ShapeDtypeStruct((NC, N, D), jnp.float32),
    mesh=_mesh,
    compiler_params=_sc_params,
    scratch_types=[
        pltpu.VMEM((IDXB, CHUNK), jnp.int32),
        pltpu.VMEM((IDXB, CHUNK), jnp.int32),
        pltpu.VMEM((2, CHUNK, D), jnp.float32),
        pltpu.VMEM_SHARED((N, D), jnp.float32),
        pltpu.SemaphoreType.DMA((2,)),
        pltpu.SemaphoreType.DMA((2,)),
    ],
)
def _sc_agg2(h_hbm, src_hbm, dst_hbm, zx_hbm, oh_hbm,
             src_v, dst_v, hrows, acch, sem_g, sem_s):
    cid = lax.axis_index("c")
    sid = lax.axis_index("s")

    def zinit(r0, nr):
        pltpu.sync_copy(zx_hbm.at[pl.ds(r0, nr)], acch.at[pl.ds(r0, nr)])

    _stripes(zinit)
    plsc.subcore_barrier()

    gb, ge = _grange(cid, sid)

    def grp(g, carry):
        pltpu.sync_copy(src_hbm.at[g], src_v)
        pltpu.sync_copy(dst_hbm.at[g], dst_v)

        def G(jj):
            return pltpu.async_copy(h_hbm.at[src_v.at[jj]],
                                    hrows.at[jj % 2], sem_g.at[jj % 2])

        def S(jj):
            return pltpu.async_copy(hrows.at[jj % 2], acch.at[dst_v.at[jj]],
                                    sem_s.at[jj % 2], add=True)

        gd0 = G(0)
        gd1 = G(1)
        gd0.wait(); sx0 = S(0)
        gd1.wait(); sx1 = S(1)
        sx0.wait(); gd2 = G(2)
        sx1.wait(); gd3 = G(3)
        gd2.wait(); sx2 = S(2)
        gd3.wait(); sx3 = S(3)
        sx2.wait(); sx3.wait()
        return carry

    lax.fori_loop(gb, ge, grp, 0)
    plsc.subcore_barrier()

    def wback(r0, nr):
        pltpu.sync_copy(acch.at[pl.ds(r0, nr)], oh_hbm.at[cid, pl.ds(r0, nr)])

    _stripes(wback)


@functools.partial(
    pl.kernel,
    out_type=jax.ShapeDtypeStruct((E, 2), jnp.float32),
    mesh=_mesh,
    compiler_params=_sc_params,
    scratch_types=[
        pltpu.VMEM((IDXB, CHUNK), jnp.int32),
        pltpu.VMEM((IDXB, CHUNK), jnp.int32),
        pltpu.VMEM((IDXB, CHUNK, 8), jnp.float32),
        pltpu.SemaphoreType.DMA((2,)),
    ],
)
def _sc_edge_score(ps_hbm, pd_hbm, src_hbm, dst_hbm, out_hbm,
                   src_v, dst_v, s_v, sem):
    cid = lax.axis_index("c")
    sid = lax.axis_index("s")
    gb, ge = _grange(cid, sid)

    def grp(g, carry):
        pltpu.sync_copy(src_hbm.at[g], src_v)
        pltpu.sync_copy(dst_hbm.at[g], dst_v)
        goff = g * GEDGE

        ds = [pltpu.async_copy(ps_hbm.at[src_v.at[jj]], s_v.at[jj],
                               sem.at[0]) for jj in range(IDXB)]
        das = []
        for jj in range(IDXB):
            ds[jj].wait()
            das.append(pltpu.async_copy(pd_hbm.at[dst_v.at[jj]], s_v.at[jj],
                                        sem.at[1], add=True))
        for jj in range(IDXB):
            das[jj].wait()
            pltpu.sync_copy(s_v.at[jj, slice(None), pl.ds(0, 2)],
                            out_hbm.at[pl.ds(goff + jj * CHUNK, CHUNK)])
        return carry

    lax.fori_loop(gb, ge, grp, 0)


RB = 400
NRB = N // RB


def _row3_spec(c, k):
    return pl.BlockSpec((1, RB, c), lambda i, k=k: (k, i, 0))


def _row_spec(c):
    return pl.BlockSpec((RB, c), lambda i: (i, 0))


def _full_spec(r, c):
    return pl.BlockSpec((r, c), lambda i: (0, 0))


def _tc_layer(ox, oe, od, h, wmh, wme, bm, wah, wan, ba, wp=None, bp=None):
    with_p = wp is not None
    if not with_p:
        wp = jnp.zeros((D, 16), jnp.float32)
        bp = jnp.zeros((1, 8), jnp.float32)

    def body(ox0r, ox1r, oe0r, oe1r, od0r, od1r, hr, wmhr, wmer, bmr, wahr,
             wanr, bar, wpr, bpr, hor, *maybe_p):
        aggh = ox0r[0] + ox1r[0]
        agge = oe0r[0] + oe1r[0]
        deg = od0r[0][:, 0:1] + od1r[0][:, 0:1]
        s = (jnp.dot(aggh, wmhr[...], preferred_element_type=jnp.float32)
             + jnp.dot(agge, wmer[...], preferred_element_type=jnp.float32)
             + deg * bmr[...])
        hn = jnp.where(deg > 0, s / jnp.maximum(deg, 1.0), 0.0)
        hnew = jax.nn.relu(
            jnp.dot(hr[...], wahr[...], preferred_element_type=jnp.float32)
            + jnp.dot(hn, wanr[...], preferred_element_type=jnp.float32)
            + bar[...])
        hor[...] = hnew
        if maybe_p:
            p = jnp.dot(hnew, wpr[...], preferred_element_type=jnp.float32)
            maybe_p[0][...] = p[:, 0:8]
            maybe_p[1][...] = p[:, 8:16] + bpr[...]

    out_shape = [jax.ShapeDtypeStruct((N, D), jnp.float32)]
    out_specs = [_row_spec(D)]
    if with_p:
        out_shape += [jax.ShapeDtypeStruct((N, 8), jnp.float32)] * 2
        out_specs += [_row_spec(8)] * 2

    res = pl.pallas_call(
        body,
        grid=(NRB,),
        in_specs=[
            _row3_spec(D, 0), _row3_spec(D, 1),
            _row3_spec(DE, 0), _row3_spec(DE, 1),
            _row3_spec(8, 0), _row3_spec(8, 1),
            _row_spec(D),
            _full_spec(D, D), _full_spec(DE, D), _full_spec(1, D),
            _full_spec(D, D), _full_spec(D, D), _full_spec(1, D),
            _full_spec(D, 16), _full_spec(1, 8),
        ],
        out_specs=out_specs,
        out_shape=out_shape,
    )(ox, ox, oe, oe, od, od, h, wmh, wme, bm, wah, wan, ba, wp, bp)
    return res if with_p else res[0]


def kernel(x, edge_index, edge_attr, W_msg1, b_msg1, W_apply1, b_apply1,
           W_msg2, b_msg2, W_apply2, b_apply2, W_pred, b_pred):
    src = edge_index[0].astype(jnp.int32)
    dst = edge_index[1].astype(jnp.int32)

    src_p = src.reshape(NGT, IDXB, CHUNK)
    dst_p = dst.reshape(NGT, IDXB, CHUNK)

    ones8 = jnp.ones((CHUNK, 8), jnp.float32)
    zx = jnp.zeros((N, D), jnp.float32)
    ze = jnp.zeros((N, DE), jnp.float32)
    zd = jnp.zeros((N, 8), jnp.float32)

    ox, oe, od = _sc_agg1(x, edge_attr, src_p, dst_p, ones8, zx, ze, zd)

    h1 = _tc_layer(ox, oe, od, x,
                   W_msg1[:D], W_msg1[D:], b_msg1[None, :],
                   W_apply1[:D], W_apply1[D:], b_apply1[None, :])

    oh = _sc_agg2(h1, src_p, dst_p, zx)

    wp16 = jnp.zeros((D, 16), jnp.float32)
    wp16 = wp16.at[:, 0:2].set(W_pred[:D]).at[:, 8:10].set(W_pred[D:])
    bp8 = jnp.zeros((1, 8), jnp.float32).at[0, 0:2].set(b_pred)

    _, ps_tab, pd_tab = _tc_layer(oh, oe, od, h1,
                                  W_msg2[:D], W_msg2[D:], b_msg2[None, :],
                                  W_apply2[:D], W_apply2[D:], b_apply2[None, :],
                                  wp=wp16, bp=bp8)

    return _sc_edge_score(ps_tab, pd_tab, src_p, dst_p)


# speedup vs baseline: 2.0158x; 1.4651x over previous
import functools

import jax
import jax.numpy as jnp
from jax import lax
from jax.experimental import pallas as pl
from jax.experimental.pallas import tpu as pltpu
from jax.experimental.pallas import tpu_sc as plsc

N = 10000
E = 320000
D = 128
DE = 16
NC = 2
NS = 16
NW = NC * NS
CHUNK = 64
IDXB = 4
GEDGE = IDXB * CHUNK
NGT = E // GEDGE
S_FULL = 632
S_LAST = N - 15 * S_FULL

_mesh = plsc.VectorSubcoreMesh(core_axis_name="c", subcore_axis_name="s")
_sc_params = pltpu.CompilerParams(use_tc_tiling_on_sc=False)


def _grange(cid, sid):
    w = cid * NS + sid
    gb = (w * NGT) // NW
    ge = ((w + 1) * NGT) // NW
    return gb, ge


def _stripes(copy_fn):
    sid = lax.axis_index("s")

    @pl.when(sid < 15)
    def _():
        copy_fn(sid * S_FULL, S_FULL)

    @pl.when(sid == 15)
    def _():
        copy_fn(15 * S_FULL, S_LAST)


@functools.partial(
    pl.kernel,
    out_type=(
        jax.ShapeDtypeStruct((NC, N, D), jnp.float32),
        jax.ShapeDtypeStruct((NC, N, DE), jnp.float32),
        jax.ShapeDtypeStruct((NC, N, 8), jnp.float32),
    ),
    mesh=_mesh,
    compiler_params=_sc_params,
    scratch_types=[
        pltpu.VMEM((IDXB, CHUNK), jnp.int32),
        pltpu.VMEM((IDXB, CHUNK), jnp.int32),
        pltpu.VMEM((2, CHUNK, D), jnp.float32),
        pltpu.VMEM((GEDGE, DE), jnp.float32),
        pltpu.VMEM((CHUNK, 8), jnp.float32),
        pltpu.VMEM_SHARED((N, D), jnp.float32),
        pltpu.VMEM_SHARED((N, DE), jnp.float32),
        pltpu.VMEM_SHARED((N, 8), jnp.float32),
        pltpu.SemaphoreType.DMA((2,)),
        pltpu.SemaphoreType.DMA((2,)),
        pltpu.SemaphoreType.DMA((2,)),
    ],
)
def _sc_agg1(x_hbm, ea_hbm, src_hbm, dst_hbm, ones_hbm, zx_hbm, ze_hbm, zd_hbm,
             ox_hbm, oe_hbm, od_hbm,
             src_v, dst_v, xrows, ea4, ones_v, accx, acce, accd,
             sem_g, sem_s, sem_e):
    cid = lax.axis_index("c")
    sid = lax.axis_index("s")

    def zinit(r0, nr):
        pltpu.sync_copy(zx_hbm.at[pl.ds(r0, nr)], accx.at[pl.ds(r0, nr)])
        pltpu.sync_copy(ze_hbm.at[pl.ds(r0, nr)], acce.at[pl.ds(r0, nr)])
        pltpu.sync_copy(zd_hbm.at[pl.ds(r0, nr)], accd.at[pl.ds(r0, nr)])

    _stripes(zinit)
    pltpu.sync_copy(ones_hbm, ones_v)
    plsc.subcore_barrier()

    gb, ge = _grange(cid, sid)

    def grp(g, carry):
        pltpu.sync_copy(src_hbm.at[g], src_v)
        pltpu.sync_copy(dst_hbm.at[g], dst_v)
        goff = g * GEDGE
        pltpu.sync_copy(ea_hbm.at[pl.ds(goff, GEDGE)], ea4)

        eds = []
        for jj in range(IDXB):
            didx = dst_v.at[jj]
            eds.append(pltpu.async_copy(
                ea4.at[pl.ds(jj * CHUNK, CHUNK)], acce.at[didx],
                sem_e.at[0], add=True))
            eds.append(pltpu.async_copy(
                ones_v, accd.at[didx], sem_e.at[1], add=True))

        def G(jj):
            return pltpu.async_copy(x_hbm.at[src_v.at[jj]],
                                    xrows.at[jj % 2], sem_g.at[jj % 2])

        def S(jj):
            return pltpu.async_copy(xrows.at[jj % 2], accx.at[dst_v.at[jj]],
                                    sem_s.at[jj % 2], add=True)

        gd0 = G(0)
        gd1 = G(1)
        gd0.wait(); sx0 = S(0)
        gd1.wait(); sx1 = S(1)
        sx0.wait(); gd2 = G(2)
        sx1.wait(); gd3 = G(3)
        gd2.wait(); sx2 = S(2)
        gd3.wait(); sx3 = S(3)
        sx2.wait(); sx3.wait()
        for d in eds:
            d.wait()
        return carry

    lax.fori_loop(gb, ge, grp, 0)
    plsc.subcore_barrier()

    def wback(r0, nr):
        pltpu.sync_copy(accx.at[pl.ds(r0, nr)], ox_hbm.at[cid, pl.ds(r0, nr)])
        pltpu.sync_copy(acce.at[pl.ds(r0, nr)], oe_hbm.at[cid, pl.ds(r0, nr)])
        pltpu.sync_copy(accd.at[pl.ds(r0, nr)], od_hbm.at[cid, pl.ds(r0, nr)])

    _stripes(wback)


@functools.partial(
    pl.kernel,
    out_type=jax.ShapeDtypeStruct((NC, N, D), jnp.float32),
    mesh=_mesh,
    compiler_params=_sc_params,
    scratch_types=[
        pltpu.VMEM((IDXB, CHUNK), jnp.int32),
        pltpu.VMEM((IDXB, CHUNK), jnp.int32),
        pltpu.VMEM((2, CHUNK, D), jnp.float32),
        pltpu.VMEM_SHARED((N, D), jnp.float32),
        pltpu.SemaphoreType.DMA((2,)),
        pltpu.SemaphoreType.DMA((2,)),
    ],
)
def _sc_agg2(h_hbm, src_hbm, dst_hbm, zx_hbm, oh_hbm,
             src_v, dst_v, hrows, acch, sem_g, sem_s):
    cid = lax.axis_index("c")
    sid = lax.axis_index("s")

    def zinit(r0, nr):
        pltpu.sync_copy(zx_hbm.at[pl.ds(r0, nr)], acch.at[pl.ds(r0, nr)])

    _stripes(zinit)
    plsc.subcore_barrier()

    gb, ge = _grange(cid, sid)

    def grp(g, carry):
        pltpu.sync_copy(src_hbm.at[g], src_v)
        pltpu.sync_copy(dst_hbm.at[g], dst_v)

        def G(jj):
            return pltpu.async_copy(h_hbm.at[src_v.at[jj]],
                                    hrows.at[jj % 2], sem_g.at[jj % 2])

        def S(jj):
            return pltpu.async_copy(hrows.at[jj % 2], acch.at[dst_v.at[jj]],
                                    sem_s.at[jj % 2], add=True)

        gd0 = G(0)
        gd1 = G(1)
        gd0.wait(); sx0 = S(0)
        gd1.wait(); sx1 = S(1)
        sx0.wait(); gd2 = G(2)
        sx1.wait(); gd3 = G(3)
        gd2.wait(); sx2 = S(2)
        gd3.wait(); sx3 = S(3)
        sx2.wait(); sx3.wait()
        return carry

    lax.fori_loop(gb, ge, grp, 0)
    plsc.subcore_barrier()

    def wback(r0, nr):
        pltpu.sync_copy(acch.at[pl.ds(r0, nr)], oh_hbm.at[cid, pl.ds(r0, nr)])

    _stripes(wback)


@functools.partial(
    pl.kernel,
    out_type=jax.ShapeDtypeStruct((E, 8), jnp.float32),
    mesh=_mesh,
    compiler_params=_sc_params,
    scratch_types=[
        pltpu.VMEM((IDXB, CHUNK), jnp.int32),
        pltpu.VMEM((IDXB, CHUNK), jnp.int32),
        pltpu.VMEM((IDXB, CHUNK, 8), jnp.float32),
        pltpu.SemaphoreType.DMA((2,)),
    ],
)
def _sc_edge_score(ps_hbm, pd_hbm, src_hbm, dst_hbm, out_hbm,
                   src_v, dst_v, s_v, sem):
    cid = lax.axis_index("c")
    sid = lax.axis_index("s")
    gb, ge = _grange(cid, sid)

    def grp(g, carry):
        pltpu.sync_copy(src_hbm.at[g], src_v)
        pltpu.sync_copy(dst_hbm.at[g], dst_v)
        goff = g * GEDGE

        ds = [pltpu.async_copy(ps_hbm.at[src_v.at[jj]], s_v.at[jj],
                               sem.at[0]) for jj in range(IDXB)]
        das = []
        for jj in range(IDXB):
            ds[jj].wait()
            das.append(pltpu.async_copy(pd_hbm.at[dst_v.at[jj]], s_v.at[jj],
                                        sem.at[1], add=True))
        for jj in range(IDXB):
            das[jj].wait()
            pltpu.sync_copy(s_v.at[jj],
                            out_hbm.at[pl.ds(goff + jj * CHUNK, CHUNK)])
        return carry

    lax.fori_loop(gb, ge, grp, 0)


CROWS = E // 16
CBLK = 2000


def _tc_compact(s8, sel):
    def body(sr, selr, outr):
        outr[...] = jnp.dot(sr[...], selr[...],
                            preferred_element_type=jnp.float32)

    return pl.pallas_call(
        body,
        grid=(CROWS // CBLK,),
        in_specs=[pl.BlockSpec((CBLK, 128), lambda i: (i, 0)),
                  pl.BlockSpec((128, 32), lambda i: (0, 0))],
        out_specs=pl.BlockSpec((CBLK, 32), lambda i: (i, 0)),
        out_shape=jax.ShapeDtypeStruct((CROWS, 32), jnp.float32),
    )(s8, sel)


RB = 400
NRB = N // RB


def _row3_spec(c, k):
    return pl.BlockSpec((1, RB, c), lambda i, k=k: (k, i, 0))


def _row_spec(c):
    return pl.BlockSpec((RB, c), lambda i: (i, 0))


def _full_spec(r, c):
    return pl.BlockSpec((r, c), lambda i: (0, 0))


def _tc_layer(ox, oe, od, h, wmh, wme, bm, wah, wan, ba, wp=None, bp=None):
    with_p = wp is not None
    if not with_p:
        wp = jnp.zeros((D, 16), jnp.float32)
        bp = jnp.zeros((1, 8), jnp.float32)

    def body(ox0r, ox1r, oe0r, oe1r, od0r, od1r, hr, wmhr, wmer, bmr, wahr,
             wanr, bar, wpr, bpr, hor, *maybe_p):
        aggh = ox0r[0] + ox1r[0]
        agge = oe0r[0] + oe1r[0]
        deg = od0r[0][:, 0:1] + od1r[0][:, 0:1]
        s = (jnp.dot(aggh, wmhr[...], preferred_element_type=jnp.float32)
             + jnp.dot(agge, wmer[...], preferred_element_type=jnp.float32)
             + deg * bmr[...])
        hn = jnp.where(deg > 0, s / jnp.maximum(deg, 1.0), 0.0)
        hnew = jax.nn.relu(
            jnp.dot(hr[...], wahr[...], preferred_element_type=jnp.float32)
            + jnp.dot(hn, wanr[...], preferred_element_type=jnp.float32)
            + bar[...])
        hor[...] = hnew
        if maybe_p:
            p = jnp.dot(hnew, wpr[...], preferred_element_type=jnp.float32)
            maybe_p[0][...] = p[:, 0:8]
            maybe_p[1][...] = p[:, 8:16] + bpr[...]

    out_shape = [jax.ShapeDtypeStruct((N, D), jnp.float32)]
    out_specs = [_row_spec(D)]
    if with_p:
        out_shape += [jax.ShapeDtypeStruct((N, 8), jnp.float32)] * 2
        out_specs += [_row_spec(8)] * 2

    res = pl.pallas_call(
        body,
        grid=(NRB,),
        in_specs=[
            _row3_spec(D, 0), _row3_spec(D, 1),
            _row3_spec(DE, 0), _row3_spec(DE, 1),
            _row3_spec(8, 0), _row3_spec(8, 1),
            _row_spec(D),
            _full_spec(D, D), _full_spec(DE, D), _full_spec(1, D),
            _full_spec(D, D), _full_spec(D, D), _full_spec(1, D),
            _full_spec(D, 16), _full_spec(1, 8),
        ],
        out_specs=out_specs,
        out_shape=out_shape,
    )(ox, ox, oe, oe, od, od, h, wmh, wme, bm, wah, wan, ba, wp, bp)
    return res if with_p else res[0]


def kernel(x, edge_index, edge_attr, W_msg1, b_msg1, W_apply1, b_apply1,
           W_msg2, b_msg2, W_apply2, b_apply2, W_pred, b_pred):
    src = edge_index[0].astype(jnp.int32)
    dst = edge_index[1].astype(jnp.int32)

    src_p = src.reshape(NGT, IDXB, CHUNK)
    dst_p = dst.reshape(NGT, IDXB, CHUNK)

    ones8 = jnp.ones((CHUNK, 8), jnp.float32)
    zx = jnp.zeros((N, D), jnp.float32)
    ze = jnp.zeros((N, DE), jnp.float32)
    zd = jnp.zeros((N, 8), jnp.float32)

    ox, oe, od = _sc_agg1(x, edge_attr, src_p, dst_p, ones8, zx, ze, zd)

    h1 = _tc_layer(ox, oe, od, x,
                   W_msg1[:D], W_msg1[D:], b_msg1[None, :],
                   W_apply1[:D], W_apply1[D:], b_apply1[None, :])

    oh = _sc_agg2(h1, src_p, dst_p, zx)

    wp16 = jnp.zeros((D, 16), jnp.float32)
    wp16 = wp16.at[:, 0:2].set(W_pred[:D]).at[:, 8:10].set(W_pred[D:])
    bp8 = jnp.zeros((1, 8), jnp.float32).at[0, 0:2].set(b_pred)

    _, ps_tab, pd_tab = _tc_layer(oh, oe, od, h1,
                                  W_msg2[:D], W_msg2[D:], b_msg2[None, :],
                                  W_apply2[:D], W_apply2[D:], b_apply2[None, :],
                                  wp=wp16, bp=bp8)

    s8 = _sc_edge_score(ps_tab, pd_tab, src_p, dst_p)
    sel = jnp.zeros((128, 32), jnp.float32)
    ke = jnp.arange(16)
    for c in range(2):
        sel = sel.at[8 * ke + c, 2 * ke + c].set(1.0)
    return _tc_compact(s8.reshape(CROWS, 128), sel).reshape(E, 2)
